# Initial kernel scaffold; baseline (speedup 1.0000x reference)
import jax
import jax.numpy as jnp
from jax.experimental import pallas as pl


def kernel(users, items, neg_items, U_mlp, U_mf, V_mlp, V_mf, U_mlp_g, U_mf_g, W1, b1, W2, b2):
    raise NotImplementedError("write your pallas kernel here")


# speedup vs baseline: 2.7169x; 2.7169x over previous
import functools

import jax
import jax.numpy as jnp
from jax import lax
from jax.experimental import pallas as pl
from jax.experimental.pallas import tpu as pltpu
from jax.experimental.pallas import tpu_sc as plsc

_B = 4096
_NNEG = 50
_EMB = 128
_NROWS_NEG = _B * _NNEG

_NC = 2
_NS = 16
_NW = _NC * _NS

_UB = _B // _NW
_NEG_PER_W = _NROWS_NEG // _NW
_CHUNK = 256
_NCHUNK = _NEG_PER_W // _CHUNK


def _sc_gather_all(users, items, neg_flat,
                   U_mlp, U_mf, U_mlp_g, U_mf_g, V_mlp, V_mf):
  mesh = plsc.VectorSubcoreMesh(core_axis_name="c", subcore_axis_name="s")

  out_type = (
      jax.ShapeDtypeStruct((_B, _EMB), jnp.float32),
      jax.ShapeDtypeStruct((_B, _EMB), jnp.float32),
      jax.ShapeDtypeStruct((_B, _EMB), jnp.float32),
      jax.ShapeDtypeStruct((_B, _EMB), jnp.float32),
      jax.ShapeDtypeStruct((_B, _EMB), jnp.float32),
      jax.ShapeDtypeStruct((_B, _EMB), jnp.float32),
      jax.ShapeDtypeStruct((_NROWS_NEG, _EMB), jnp.float32),
      jax.ShapeDtypeStruct((_NROWS_NEG, _EMB), jnp.float32),
  )

  @functools.partial(
      pl.kernel, mesh=mesh, out_type=out_type,
      scratch_types=[
          pltpu.VMEM((_UB,), jnp.int32),
          pltpu.VMEM((_CHUNK,), jnp.int32),
          pltpu.VMEM((_UB, _EMB), jnp.float32),
          pltpu.VMEM((_CHUNK, _EMB), jnp.float32),
          pltpu.SemaphoreType.DMA,
      ],
  )
  def k(users_h, items_h, neg_h, Um_h, Uf_h, Umg_h, Ufg_h, Vm_h, Vf_h,
        o_um, o_uf, o_umg, o_ufg, o_vm, o_vf, o_nm, o_nf,
        idx_s, idx_n, rows_s, rows_n, sem):
    wid = lax.axis_index("s") * _NC + lax.axis_index("c")
    ub = wid * _UB

    pltpu.sync_copy(users_h.at[pl.ds(ub, _UB)], idx_s)
    for tbl, out in ((Um_h, o_um), (Uf_h, o_uf), (Umg_h, o_umg), (Ufg_h, o_ufg)):
      pltpu.async_copy(tbl.at[idx_s], rows_s, sem).wait()
      pltpu.sync_copy(rows_s, out.at[pl.ds(ub, _UB)])

    pltpu.sync_copy(items_h.at[pl.ds(ub, _UB)], idx_s)
    for tbl, out in ((Vm_h, o_vm), (Vf_h, o_vf)):
      pltpu.async_copy(tbl.at[idx_s], rows_s, sem).wait()
      pltpu.sync_copy(rows_s, out.at[pl.ds(ub, _UB)])

    nb = wid * _NEG_PER_W

    def body(c, carry):
      base = nb + c * _CHUNK
      pltpu.sync_copy(neg_h.at[pl.ds(base, _CHUNK)], idx_n)
      pltpu.async_copy(Vm_h.at[idx_n], rows_n, sem).wait()
      pltpu.sync_copy(rows_n, o_nm.at[pl.ds(base, _CHUNK)])
      pltpu.async_copy(Vf_h.at[idx_n], rows_n, sem).wait()
      pltpu.sync_copy(rows_n, o_nf.at[pl.ds(base, _CHUNK)])
      return carry

    lax.fori_loop(0, _NCHUNK, body, 0)

  return k(users, items, neg_flat, U_mlp, U_mf, U_mlp_g, U_mf_g, V_mlp, V_mf)


_BU = 32
_NBLK = _BU * _NNEG


def _tc_math_kernel(u_mlp_r, v_mlp_r, u_mf_r, v_mf_r, nvm_r, nvf_r,
                    W1u_r, W1v_r, b1_r, W2_r, b2_r,
                    o_mlp, o_mf, o_nmlp, o_nmf):
  f32 = jnp.float32
  u = u_mlp_r[...]
  pre_u = jax.lax.dot_general(u, W1u_r[...], (((1,), (0,)), ((), ())),
                              preferred_element_type=f32)
  pre_v = jax.lax.dot_general(v_mlp_r[...], W1v_r[...], (((1,), (0,)), ((), ())),
                              preferred_element_type=f32)
  b1 = b1_r[...]
  b2 = b2_r[...]
  h = jax.nn.relu(pre_u + pre_v + b1)
  o_mlp[...] = jax.lax.dot_general(h, W2_r[...], (((1,), (0,)), ((), ())),
                                   preferred_element_type=f32) + b2
  o_mf[...] = u_mf_r[...] * v_mf_r[...]

  pre_nv = jax.lax.dot_general(nvm_r[...], W1v_r[...], (((1,), (0,)), ((), ())),
                               preferred_element_type=f32)
  pre_u_rep = jnp.repeat(pre_u, _NNEG, axis=0)
  hn = jax.nn.relu(pre_u_rep + pre_nv + b1)
  o_nmlp[...] = jax.lax.dot_general(hn, W2_r[...], (((1,), (0,)), ((), ())),
                                    preferred_element_type=f32) + b2
  o_nmf[...] = jnp.repeat(u_mf_r[...], _NNEG, axis=0) * nvf_r[...]


def _tc_math(u_mlp, v_mlp, u_mf, v_mf, neg_v_mlp, neg_v_mf,
             W1u, W1v, b1, W2, b2):
  grid = (_B // _BU,)
  blk = lambda r: pl.BlockSpec((r, _EMB), lambda i: (i, 0))
  rep = lambda r: pl.BlockSpec((r, _EMB), lambda i: (0, 0))
  out = pl.pallas_call(
      _tc_math_kernel,
      grid=grid,
      in_specs=[
          blk(_BU), blk(_BU), blk(_BU), blk(_BU),
          blk(_NBLK), blk(_NBLK),
          rep(_EMB), rep(_EMB),
          rep(1),
          rep(_EMB),
          rep(1),
      ],
      out_specs=[blk(_BU), blk(_BU), blk(_NBLK), blk(_NBLK)],
      out_shape=[
          jax.ShapeDtypeStruct((_B, _EMB), jnp.float32),
          jax.ShapeDtypeStruct((_B, _EMB), jnp.float32),
          jax.ShapeDtypeStruct((_NROWS_NEG, _EMB), jnp.float32),
          jax.ShapeDtypeStruct((_NROWS_NEG, _EMB), jnp.float32),
      ],
  )(u_mlp, v_mlp, u_mf, v_mf, neg_v_mlp, neg_v_mf, W1u, W1v, b1, W2, b2)
  return out


def kernel(users, items, neg_items, U_mlp, U_mf, V_mlp, V_mf,
           U_mlp_g, U_mf_g, W1, b1, W2, b2):
  users = users.astype(jnp.int32)
  items = items.astype(jnp.int32)
  neg_flat = neg_items.astype(jnp.int32).reshape(-1)

  (u_mlp, u_mf, u_mlp_g, u_mf_g, v_mlp, v_mf,
   neg_v_mlp, neg_v_mf) = _sc_gather_all(
      users, items, neg_flat, U_mlp, U_mf, U_mlp_g, U_mf_g, V_mlp, V_mf)

  W1u = W1[:_EMB]
  W1v = W1[_EMB:]
  b1r = b1.reshape(1, _EMB)
  b2r = b2.reshape(1, _EMB)

  mlp_vec, mf_vec, neg_mlp, neg_mf = _tc_math(
      u_mlp, v_mlp, u_mf, v_mf, neg_v_mlp, neg_v_mf, W1u, W1v, b1r, W2, b2r)

  return (mlp_vec, mf_vec, u_mlp, u_mf, u_mlp_g, u_mf_g,
          neg_mlp.reshape(_B, _NNEG, _EMB), neg_mf.reshape(_B, _NNEG, _EMB))


# speedup vs baseline: 2.7191x; 1.0008x over previous
import functools

import jax
import jax.numpy as jnp
from jax import lax
from jax.experimental import pallas as pl
from jax.experimental.pallas import tpu as pltpu
from jax.experimental.pallas import tpu_sc as plsc

_B = 4096
_NNEG = 50
_EMB = 128
_NROWS_NEG = _B * _NNEG

_NC = 2
_NS = 16
_NW = _NC * _NS

_UB = _B // _NW
_NEG_PER_W = _NROWS_NEG // _NW
_CHUNK = 256
_NCHUNK = _NEG_PER_W // _CHUNK


def _sc_gather_all(users, items, neg_flat,
                   U_mlp, U_mf, U_mlp_g, U_mf_g, V_mlp, V_mf):
  mesh = plsc.VectorSubcoreMesh(core_axis_name="c", subcore_axis_name="s")

  out_type = (
      jax.ShapeDtypeStruct((_B, _EMB), jnp.float32),
      jax.ShapeDtypeStruct((_B, _EMB), jnp.float32),
      jax.ShapeDtypeStruct((_B, _EMB), jnp.float32),
      jax.ShapeDtypeStruct((_B, _EMB), jnp.float32),
      jax.ShapeDtypeStruct((_B, _EMB), jnp.float32),
      jax.ShapeDtypeStruct((_B, _EMB), jnp.float32),
      jax.ShapeDtypeStruct((_NROWS_NEG, _EMB), jnp.float32),
      jax.ShapeDtypeStruct((_NROWS_NEG, _EMB), jnp.float32),
  )

  @functools.partial(
      pl.kernel, mesh=mesh, out_type=out_type,
      compiler_params=pltpu.CompilerParams(use_tc_tiling_on_sc=True),
      scratch_types=[
          pltpu.VMEM((_UB,), jnp.int32),
          pltpu.VMEM((_CHUNK,), jnp.int32),
          pltpu.VMEM((_UB, _EMB), jnp.float32),
          pltpu.VMEM((_CHUNK, _EMB), jnp.float32),
          pltpu.SemaphoreType.DMA,
      ],
  )
  def k(users_h, items_h, neg_h, Um_h, Uf_h, Umg_h, Ufg_h, Vm_h, Vf_h,
        o_um, o_uf, o_umg, o_ufg, o_vm, o_vf, o_nm, o_nf,
        idx_s, idx_n, rows_s, rows_n, sem):
    wid = lax.axis_index("s") * _NC + lax.axis_index("c")
    ub = wid * _UB

    pltpu.sync_copy(users_h.at[pl.ds(ub, _UB)], idx_s)
    for tbl, out in ((Um_h, o_um), (Uf_h, o_uf), (Umg_h, o_umg), (Ufg_h, o_ufg)):
      pltpu.async_copy(tbl.at[idx_s], rows_s, sem).wait()
      pltpu.sync_copy(rows_s, out.at[pl.ds(ub, _UB)])

    pltpu.sync_copy(items_h.at[pl.ds(ub, _UB)], idx_s)
    for tbl, out in ((Vm_h, o_vm), (Vf_h, o_vf)):
      pltpu.async_copy(tbl.at[idx_s], rows_s, sem).wait()
      pltpu.sync_copy(rows_s, out.at[pl.ds(ub, _UB)])

    nb = wid * _NEG_PER_W

    def body(c, carry):
      base = nb + c * _CHUNK
      pltpu.sync_copy(neg_h.at[pl.ds(base, _CHUNK)], idx_n)
      pltpu.async_copy(Vm_h.at[idx_n], rows_n, sem).wait()
      pltpu.sync_copy(rows_n, o_nm.at[pl.ds(base, _CHUNK)])
      pltpu.async_copy(Vf_h.at[idx_n], rows_n, sem).wait()
      pltpu.sync_copy(rows_n, o_nf.at[pl.ds(base, _CHUNK)])
      return carry

    lax.fori_loop(0, _NCHUNK, body, 0)

  return k(users, items, neg_flat, U_mlp, U_mf, U_mlp_g, U_mf_g, V_mlp, V_mf)


_BU = 32
_NBLK = _BU * _NNEG


def _tc_math_kernel(u_mlp_r, v_mlp_r, u_mf_r, v_mf_r, nvm_r, nvf_r,
                    W1u_r, W1v_r, b1_r, W2_r, b2_r,
                    o_mlp, o_mf, o_nmlp, o_nmf):
  f32 = jnp.float32
  bf16 = jnp.bfloat16
  dot = lambda a, b: jax.lax.dot_general(
      a.astype(bf16), b, (((1,), (0,)), ((), ())), preferred_element_type=f32)
  W1u = W1u_r[...].astype(bf16)
  W1v = W1v_r[...].astype(bf16)
  W2 = W2_r[...].astype(bf16)
  pre_u = dot(u_mlp_r[...], W1u)
  pre_v = dot(v_mlp_r[...], W1v)
  b1 = b1_r[...]
  b2 = b2_r[...]
  h = jax.nn.relu(pre_u + pre_v + b1)
  o_mlp[...] = dot(h, W2) + b2
  o_mf[...] = u_mf_r[...] * v_mf_r[...]

  pre_nv = dot(nvm_r[...], W1v)
  pre_u_rep = jnp.repeat(pre_u, _NNEG, axis=0)
  hn = jax.nn.relu(pre_u_rep + pre_nv + b1)
  o_nmlp[...] = dot(hn, W2) + b2
  o_nmf[...] = jnp.repeat(u_mf_r[...], _NNEG, axis=0) * nvf_r[...]


def _tc_math(u_mlp, v_mlp, u_mf, v_mf, neg_v_mlp, neg_v_mf,
             W1u, W1v, b1, W2, b2):
  grid = (_B // _BU,)
  blk = lambda r: pl.BlockSpec((r, _EMB), lambda i: (i, 0))
  rep = lambda r: pl.BlockSpec((r, _EMB), lambda i: (0, 0))
  out = pl.pallas_call(
      _tc_math_kernel,
      grid=grid,
      in_specs=[
          blk(_BU), blk(_BU), blk(_BU), blk(_BU),
          blk(_NBLK), blk(_NBLK),
          rep(_EMB), rep(_EMB),
          rep(1),
          rep(_EMB),
          rep(1),
      ],
      out_specs=[blk(_BU), blk(_BU), blk(_NBLK), blk(_NBLK)],
      out_shape=[
          jax.ShapeDtypeStruct((_B, _EMB), jnp.float32),
          jax.ShapeDtypeStruct((_B, _EMB), jnp.float32),
          jax.ShapeDtypeStruct((_NROWS_NEG, _EMB), jnp.float32),
          jax.ShapeDtypeStruct((_NROWS_NEG, _EMB), jnp.float32),
      ],
  )(u_mlp, v_mlp, u_mf, v_mf, neg_v_mlp, neg_v_mf, W1u, W1v, b1, W2, b2)
  return out


def kernel(users, items, neg_items, U_mlp, U_mf, V_mlp, V_mf,
           U_mlp_g, U_mf_g, W1, b1, W2, b2):
  users = users.astype(jnp.int32)
  items = items.astype(jnp.int32)
  neg_flat = neg_items.astype(jnp.int32).reshape(-1)

  (u_mlp, u_mf, u_mlp_g, u_mf_g, v_mlp, v_mf,
   neg_v_mlp, neg_v_mf) = _sc_gather_all(
      users, items, neg_flat, U_mlp, U_mf, U_mlp_g, U_mf_g, V_mlp, V_mf)

  W1u = W1[:_EMB]
  W1v = W1[_EMB:]
  b1r = b1.reshape(1, _EMB)
  b2r = b2.reshape(1, _EMB)

  mlp_vec, mf_vec, neg_mlp, neg_mf = _tc_math(
      u_mlp, v_mlp, u_mf, v_mf, neg_v_mlp, neg_v_mf, W1u, W1v, b1r, W2, b2r)

  return (mlp_vec, mf_vec, u_mlp, u_mf, u_mlp_g, u_mf_g,
          neg_mlp.reshape(_B, _NNEG, _EMB), neg_mf.reshape(_B, _NNEG, _EMB))


# speedup vs baseline: 7.8594x; 2.8904x over previous
import functools

import jax
import jax.numpy as jnp
from jax import lax
from jax.experimental import pallas as pl
from jax.experimental.pallas import tpu as pltpu
from jax.experimental.pallas import tpu_sc as plsc

_B = 4096
_NNEG = 50
_EMB = 128
_NROWS_NEG = _B * _NNEG

_NC = 2
_NS = 16
_NW = _NC * _NS

_UB = _B // _NW
_NBUF = 2
_T = _NNEG // _NBUF


def _sc_gather_all(users, items, neg_flat,
                   U_mlp, U_mf, U_mlp_g, U_mf_g, V_mlp, V_mf):
  mesh = plsc.VectorSubcoreMesh(core_axis_name="c", subcore_axis_name="s")

  out_type = tuple(
      jax.ShapeDtypeStruct((_B, _EMB), jnp.float32) for _ in range(6)
  ) + (
      jax.ShapeDtypeStruct((_NROWS_NEG, _EMB), jnp.float32),
      jax.ShapeDtypeStruct((_NROWS_NEG, _EMB), jnp.float32),
  )

  scratch = (
      [
          pltpu.VMEM((_UB,), jnp.int32),
          pltpu.VMEM((_UB,), jnp.int32),
          pltpu.VMEM((_NNEG * _UB,), jnp.int32),
          pltpu.VMEM((_UB, _EMB), jnp.float32),
      ]
      + [pltpu.VMEM((_UB, _EMB), jnp.float32)] * (2 * _NBUF)
      + [pltpu.SemaphoreType.DMA] * (4 * _NBUF + 1)
  )

  def body(users_h, items_h, neg_h, Um_h, Uf_h, Umg_h, Ufg_h, Vm_h, Vf_h,
           o_um, o_uf, o_umg, o_ufg, o_vm, o_vf, o_nm, o_nf,
           idx_u, idx_i, idx_all, ubuf, *rest):
    bufA = rest[0:_NBUF]
    bufB = rest[_NBUF:2 * _NBUF]
    gA = rest[2 * _NBUF:3 * _NBUF]
    gB = rest[3 * _NBUF:4 * _NBUF]
    sA = rest[4 * _NBUF:5 * _NBUF]
    sB = rest[5 * _NBUF:6 * _NBUF]
    sem_u = rest[6 * _NBUF]

    wid = lax.axis_index("s") * _NC + lax.axis_index("c")
    b0 = wid * _UB

    pltpu.sync_copy(users_h.at[pl.ds(b0, _UB)], idx_u)
    pltpu.sync_copy(items_h.at[pl.ds(b0, _UB)], idx_i)
    pltpu.async_copy(Uf_h.at[idx_u], ubuf, sem_u)
    six = list(zip(
        (Um_h, Uf_h, Umg_h, Ufg_h, Vm_h, Vf_h),
        (o_um, o_uf, o_umg, o_ufg, o_vm, o_vf),
        (idx_u, idx_u, idx_u, idx_u, idx_i, idx_i)))
    bufs4, sems4 = bufA + bufB, gA + gB
    for lo in (0, 4):
      wave = [(t, o, i, bufs4[k], sems4[k])
              for k, (t, o, i) in enumerate(six[lo:lo + 4])]
      for tbl, _, idx, buf, sem in wave:
        pltpu.async_copy(tbl.at[idx], buf, sem)
      for tbl, _, idx, buf, sem in wave:
        pltpu.make_async_copy(tbl.at[idx], buf, sem).wait()
      for _, out, _, buf, sem in wave:
        pltpu.async_copy(buf, out.at[pl.ds(b0, _UB)], sem)
      for _, out, _, buf, sem in wave:
        pltpu.make_async_copy(buf, out.at[pl.ds(b0, _UB)], sem).wait()
    pltpu.make_async_copy(Uf_h.at[idx_u], ubuf, sem_u).wait()

    for n in range(_NNEG):
      pltpu.async_copy(neg_h.at[pl.ds(n * _B + b0, _UB)],
                       idx_all.at[pl.ds(n * _UB, _UB)], sem_u)
    for n in range(_NNEG):
      pltpu.make_async_copy(neg_h.at[pl.ds(n * _B + b0, _UB)],
                            idx_all.at[pl.ds(n * _UB, _UB)], sem_u).wait()

    def g_start(c, j):
      idx = idx_all.at[pl.ds(c * _UB, _UB)]
      pltpu.async_copy(Vm_h.at[idx], bufA[j], gA[j])
      pltpu.async_copy(Vf_h.at[idx], bufB[j], gB[j])

    def g_wait(c, j):
      idx = idx_all.at[pl.ds(c * _UB, _UB)]
      pltpu.make_async_copy(Vm_h.at[idx], bufA[j], gA[j]).wait()
      pltpu.make_async_copy(Vf_h.at[idx], bufB[j], gB[j]).wait()

    def s_start(base, j):
      pltpu.async_copy(bufA[j], o_nm.at[pl.ds(base, _UB)], sA[j])
      pltpu.async_copy(bufB[j], o_nf.at[pl.ds(base, _UB)], sB[j])

    def s_wait(base, j):
      pltpu.make_async_copy(bufA[j], o_nm.at[pl.ds(base, _UB)], sA[j]).wait()
      pltpu.make_async_copy(bufB[j], o_nf.at[pl.ds(base, _UB)], sB[j]).wait()

    def mul_rows(j):
      buf = bufB[j]
      def row(i, carry):
        for k in range(_EMB // 16):
          s = pl.ds(k * 16, 16)
          buf[i, s] = buf[i, s] * ubuf[i, s]
        return carry
      lax.fori_loop(0, _UB, row, 0)

    g_start(0, 0)

    def step(t, carry):
      for j in range(_NBUF):
        c = t * _NBUF + j
        base = c * _B + b0
        jn = (j - 1) % _NBUF
        g_wait(c, j)
        if j == 0:
          @pl.when(t > 0)
          def _():
            s_wait(base - _B, jn)
          g_start(c + 1, jn)
        else:
          s_wait(base - _B, jn)
          @pl.when(t < _T - 1)
          def _():
            g_start(c + 1, jn)
        mul_rows(j)
        s_start(base, j)
      return carry

    lax.fori_loop(0, _T, step, 0)
    s_wait((_NNEG - 1) * _B + b0, _NBUF - 1)

  return functools.partial(
      pl.kernel, mesh=mesh, out_type=out_type,
      compiler_params=pltpu.CompilerParams(use_tc_tiling_on_sc=True),
      scratch_types=scratch,
  )(body)(users, items, neg_flat, U_mlp, U_mf, U_mlp_g, U_mf_g, V_mlp, V_mf)


_BP = 512


def _tc_pos_kernel(u_mlp_r, v_mlp_r, u_mf_r, v_mf_r,
                   W1u_r, W1v_r, b1_r, W2_r, b2_r,
                   o_mlp, o_mf, o_preu):
  f32 = jnp.float32
  bf16 = jnp.bfloat16
  dot = lambda a, b: jax.lax.dot_general(
      a.astype(bf16), b, (((1,), (0,)), ((), ())), preferred_element_type=f32)
  W1u = W1u_r[...].astype(bf16)
  W1v = W1v_r[...].astype(bf16)
  W2 = W2_r[...].astype(bf16)
  pre_u = dot(u_mlp_r[...], W1u) + b1_r[...]
  o_preu[...] = pre_u
  pre_v = dot(v_mlp_r[...], W1v)
  h = jax.nn.relu(pre_u + pre_v)
  o_mlp[...] = dot(h, W2) + b2_r[...]
  o_mf[...] = u_mf_r[...] * v_mf_r[...]


def _tc_pos(u_mlp, v_mlp, u_mf, v_mf, W1u, W1v, b1, W2, b2):
  blk = pl.BlockSpec((_BP, _EMB), lambda i: (i, 0))
  rep = lambda r: pl.BlockSpec((r, _EMB), lambda i: (0, 0))
  return pl.pallas_call(
      _tc_pos_kernel,
      grid=(_B // _BP,),
      in_specs=[blk, blk, blk, blk, rep(_EMB), rep(_EMB), rep(1), rep(_EMB), rep(1)],
      out_specs=[blk, blk, blk],
      out_shape=[
          jax.ShapeDtypeStruct((_B, _EMB), jnp.float32),
          jax.ShapeDtypeStruct((_B, _EMB), jnp.float32),
          jax.ShapeDtypeStruct((_B, _EMB), jnp.float32),
      ],
  )(u_mlp, v_mlp, u_mf, v_mf, W1u, W1v, b1, W2, b2)


def _tc_neg_kernel(nvm_r, preu_r, W1v_r, W2_r, b2_r, o_nmlp):
  f32 = jnp.float32
  bf16 = jnp.bfloat16
  dot = lambda a, b: jax.lax.dot_general(
      a.astype(bf16), b, (((1,), (0,)), ((), ())), preferred_element_type=f32)
  pre_nv = dot(nvm_r[...], W1v_r[...].astype(bf16))
  h = jax.nn.relu(preu_r[...] + pre_nv)
  o_nmlp[...] = dot(h, W2_r[...].astype(bf16)) + b2_r[...]


def _tc_neg(nm, pre_u, W1v, W2, b2):
  blk = pl.BlockSpec((_B, _EMB), lambda i: (i, 0))
  ublk = pl.BlockSpec((_B, _EMB), lambda i: (0, 0))
  rep = lambda r: pl.BlockSpec((r, _EMB), lambda i: (0, 0))
  return pl.pallas_call(
      _tc_neg_kernel,
      grid=(_NNEG,),
      in_specs=[blk, ublk, rep(_EMB), rep(_EMB), rep(1)],
      out_specs=blk,
      out_shape=jax.ShapeDtypeStruct((_NROWS_NEG, _EMB), jnp.float32),
  )(nm, pre_u, W1v, W2, b2)


def kernel(users, items, neg_items, U_mlp, U_mf, V_mlp, V_mf,
           U_mlp_g, U_mf_g, W1, b1, W2, b2):
  users = users.astype(jnp.int32)
  items = items.astype(jnp.int32)
  neg_flat = neg_items.astype(jnp.int32).T.reshape(-1)

  (u_mlp, u_mf, u_mlp_g, u_mf_g, v_mlp, v_mf, nm, neg_mf_flat) = \
      _sc_gather_all(users, items, neg_flat, U_mlp, U_mf, U_mlp_g, U_mf_g,
                     V_mlp, V_mf)

  W1u = W1[:_EMB]
  W1v = W1[_EMB:]
  b1r = b1.reshape(1, _EMB)
  b2r = b2.reshape(1, _EMB)

  mlp_vec, mf_vec, pre_u = _tc_pos(
      u_mlp, v_mlp, u_mf, v_mf, W1u, W1v, b1r, W2, b2r)

  neg_mlp = _tc_neg(nm, pre_u, W1v, W2, b2r)

  neg_mlp = jnp.transpose(neg_mlp.reshape(_NNEG, _B, _EMB), (1, 0, 2))
  neg_mf = jnp.transpose(neg_mf_flat.reshape(_NNEG, _B, _EMB), (1, 0, 2))

  return (mlp_vec, mf_vec, u_mlp, u_mf, u_mlp_g, u_mf_g, neg_mlp, neg_mf)


# speedup vs baseline: 8.0003x; 1.0179x over previous
import functools

import jax
import jax.numpy as jnp
from jax import lax
from jax.experimental import pallas as pl
from jax.experimental.pallas import tpu as pltpu
from jax.experimental.pallas import tpu_sc as plsc

_B = 4096
_NNEG = 50
_EMB = 128
_NROWS_NEG = _B * _NNEG

_NC = 2
_NS = 16
_NW = _NC * _NS

_UB = _B // _NW
_NBUF = 2
_T = _NNEG // _NBUF


def _sc_gather_all(users, items, neg_flat,
                   U_mlp, U_mf, U_mlp_g, U_mf_g, V_mlp, V_mf):
  mesh = plsc.VectorSubcoreMesh(core_axis_name="c", subcore_axis_name="s")

  out_type = tuple(
      jax.ShapeDtypeStruct((_B, _EMB), jnp.float32) for _ in range(6)
  ) + (
      jax.ShapeDtypeStruct((_NROWS_NEG, _EMB), jnp.float32),
      jax.ShapeDtypeStruct((_NROWS_NEG, _EMB), jnp.float32),
  )

  scratch = (
      [
          pltpu.VMEM((_UB,), jnp.int32),
          pltpu.VMEM((_UB,), jnp.int32),
          pltpu.VMEM((_NNEG * _UB,), jnp.int32),
          pltpu.VMEM((_UB, _EMB), jnp.float32),
      ]
      + [pltpu.VMEM((_UB, _EMB), jnp.float32)] * (2 * _NBUF)
      + [pltpu.SemaphoreType.DMA] * (4 * _NBUF + 1)
  )

  def body(users_h, items_h, neg_h, Um_h, Uf_h, Umg_h, Ufg_h, Vm_h, Vf_h,
           o_um, o_uf, o_umg, o_ufg, o_vm, o_vf, o_nm, o_nf,
           idx_u, idx_i, idx_all, ubuf, *rest):
    bufA = rest[0:_NBUF]
    bufB = rest[_NBUF:2 * _NBUF]
    gA = rest[2 * _NBUF:3 * _NBUF]
    gB = rest[3 * _NBUF:4 * _NBUF]
    sA = rest[4 * _NBUF:5 * _NBUF]
    sB = rest[5 * _NBUF:6 * _NBUF]
    sem_u = rest[6 * _NBUF]

    wid = lax.axis_index("s") * _NC + lax.axis_index("c")
    b0 = wid * _UB

    pltpu.sync_copy(users_h.at[pl.ds(b0, _UB)], idx_u)
    pltpu.sync_copy(items_h.at[pl.ds(b0, _UB)], idx_i)
    pltpu.async_copy(Uf_h.at[idx_u], ubuf, sem_u)
    six = list(zip(
        (Um_h, Uf_h, Umg_h, Ufg_h, Vm_h, Vf_h),
        (o_um, o_uf, o_umg, o_ufg, o_vm, o_vf),
        (idx_u, idx_u, idx_u, idx_u, idx_i, idx_i)))
    bufs4, sems4 = bufA + bufB, gA + gB
    for lo in (0, 4):
      wave = [(t, o, i, bufs4[k], sems4[k])
              for k, (t, o, i) in enumerate(six[lo:lo + 4])]
      for tbl, _, idx, buf, sem in wave:
        pltpu.async_copy(tbl.at[idx], buf, sem)
      for tbl, _, idx, buf, sem in wave:
        pltpu.make_async_copy(tbl.at[idx], buf, sem).wait()
      for _, out, _, buf, sem in wave:
        pltpu.async_copy(buf, out.at[pl.ds(b0, _UB)], sem)
      for _, out, _, buf, sem in wave:
        pltpu.make_async_copy(buf, out.at[pl.ds(b0, _UB)], sem).wait()
    pltpu.make_async_copy(Uf_h.at[idx_u], ubuf, sem_u).wait()

    for n in range(_NNEG):
      pltpu.async_copy(neg_h.at[pl.ds(n * _B + b0, _UB)],
                       idx_all.at[pl.ds(n * _UB, _UB)], sem_u)
    for n in range(_NNEG):
      pltpu.make_async_copy(neg_h.at[pl.ds(n * _B + b0, _UB)],
                            idx_all.at[pl.ds(n * _UB, _UB)], sem_u).wait()

    def g_start(c, j):
      idx = idx_all.at[pl.ds(c * _UB, _UB)]
      pltpu.async_copy(Vm_h.at[idx], bufA[j], gA[j])
      pltpu.async_copy(Vf_h.at[idx], bufB[j], gB[j])

    def g_wait(c, j):
      idx = idx_all.at[pl.ds(c * _UB, _UB)]
      pltpu.make_async_copy(Vm_h.at[idx], bufA[j], gA[j]).wait()
      pltpu.make_async_copy(Vf_h.at[idx], bufB[j], gB[j]).wait()

    def s_start(base, j):
      pltpu.async_copy(bufA[j], o_nm.at[pl.ds(base, _UB)], sA[j])
      pltpu.async_copy(bufB[j], o_nf.at[pl.ds(base, _UB)], sB[j])

    def s_wait(base, j):
      pltpu.make_async_copy(bufA[j], o_nm.at[pl.ds(base, _UB)], sA[j]).wait()
      pltpu.make_async_copy(bufB[j], o_nf.at[pl.ds(base, _UB)], sB[j]).wait()

    def mul_rows(j):
      buf = bufB[j]
      def row(i, carry):
        for k in range(_EMB // 16):
          s = pl.ds(k * 16, 16)
          buf[i, s] = buf[i, s] * ubuf[i, s]
        return carry
      lax.fori_loop(0, _UB, row, 0)

    g_start(0, 0)

    def step(t, carry):
      for j in range(_NBUF):
        c = t * _NBUF + j
        base = c * _B + b0
        jn = (j - 1) % _NBUF
        g_wait(c, j)
        if j == 0:
          @pl.when(t > 0)
          def _():
            s_wait(base - _B, jn)
          g_start(c + 1, jn)
        else:
          s_wait(base - _B, jn)
          @pl.when(t < _T - 1)
          def _():
            g_start(c + 1, jn)
        mul_rows(j)
        s_start(base, j)
      return carry

    lax.fori_loop(0, _T, step, 0)
    s_wait((_NNEG - 1) * _B + b0, _NBUF - 1)

  return functools.partial(
      pl.kernel, mesh=mesh, out_type=out_type,
      compiler_params=pltpu.CompilerParams(use_tc_tiling_on_sc=True),
      scratch_types=scratch,
  )(body)(users, items, neg_flat, U_mlp, U_mf, U_mlp_g, U_mf_g, V_mlp, V_mf)


def _tc_mlp_kernel(u_mlp_r, v_mlp_r, u_mf_r, v_mf_r, nvm_r,
                   W1u_r, W1v_r, b1_r, W2_r, b2_r,
                   o_mlp, o_mf, o_nmlp, preu_s):
  f32 = jnp.float32
  bf16 = jnp.bfloat16
  dot = lambda a, b: jax.lax.dot_general(
      a.astype(bf16), b, (((1,), (0,)), ((), ())), preferred_element_type=f32)
  i = pl.program_id(0)
  W1v = W1v_r[...].astype(bf16)
  W2 = W2_r[...].astype(bf16)

  @pl.when(i == 0)
  def _():
    pre_u = dot(u_mlp_r[...], W1u_r[...].astype(bf16)) + b1_r[...]
    preu_s[...] = pre_u
    pre_v = dot(v_mlp_r[...], W1v)
    h = jax.nn.relu(pre_u + pre_v)
    o_mlp[...] = dot(h, W2) + b2_r[...]
    o_mf[...] = u_mf_r[...] * v_mf_r[...]

  @pl.when(i > 0)
  def _():
    pre_nv = dot(nvm_r[...], W1v)
    h = jax.nn.relu(preu_s[...] + pre_nv)
    o_nmlp[...] = dot(h, W2) + b2_r[...]


def _tc_mlp(u_mlp, v_mlp, u_mf, v_mf, nm, W1u, W1v, b1, W2, b2):
  blk0 = pl.BlockSpec((_B, _EMB), lambda i: (0, 0))
  nblk = pl.BlockSpec((_B, _EMB), lambda i: (jnp.maximum(i - 1, 0), 0))
  rep = lambda r: pl.BlockSpec((r, _EMB), lambda i: (0, 0))
  return pl.pallas_call(
      _tc_mlp_kernel,
      grid=(_NNEG + 1,),
      in_specs=[blk0, blk0, blk0, blk0, nblk,
                rep(_EMB), rep(_EMB), rep(1), rep(_EMB), rep(1)],
      out_specs=[blk0, blk0, nblk],
      out_shape=[
          jax.ShapeDtypeStruct((_B, _EMB), jnp.float32),
          jax.ShapeDtypeStruct((_B, _EMB), jnp.float32),
          jax.ShapeDtypeStruct((_NROWS_NEG, _EMB), jnp.float32),
      ],
      scratch_shapes=[pltpu.VMEM((_B, _EMB), jnp.float32)],
  )(u_mlp, v_mlp, u_mf, v_mf, nm, W1u, W1v, b1, W2, b2)


def kernel(users, items, neg_items, U_mlp, U_mf, V_mlp, V_mf,
           U_mlp_g, U_mf_g, W1, b1, W2, b2):
  users = users.astype(jnp.int32)
  items = items.astype(jnp.int32)
  neg_flat = neg_items.astype(jnp.int32).T.reshape(-1)

  (u_mlp, u_mf, u_mlp_g, u_mf_g, v_mlp, v_mf, nm, neg_mf_flat) = \
      _sc_gather_all(users, items, neg_flat, U_mlp, U_mf, U_mlp_g, U_mf_g,
                     V_mlp, V_mf)

  W1u = W1[:_EMB]
  W1v = W1[_EMB:]
  b1r = b1.reshape(1, _EMB)
  b2r = b2.reshape(1, _EMB)

  mlp_vec, mf_vec, neg_mlp = _tc_mlp(
      u_mlp, v_mlp, u_mf, v_mf, nm, W1u, W1v, b1r, W2, b2r)

  neg_mlp = jnp.transpose(neg_mlp.reshape(_NNEG, _B, _EMB), (1, 0, 2))
  neg_mf = jnp.transpose(neg_mf_flat.reshape(_NNEG, _B, _EMB), (1, 0, 2))

  return (mlp_vec, mf_vec, u_mlp, u_mf, u_mlp_g, u_mf_g, neg_mlp, neg_mf)


# speedup vs baseline: 8.0417x; 1.0052x over previous
import functools

import jax
import jax.numpy as jnp
from jax import lax
from jax.experimental import pallas as pl
from jax.experimental.pallas import tpu as pltpu
from jax.experimental.pallas import tpu_sc as plsc

_B = 4096
_NNEG = 50
_EMB = 128
_NROWS_NEG = _B * _NNEG

_NC = 2
_NS = 16
_NW = _NC * _NS

_UB = _B // _NW
_NBUF = 2
_T = _NNEG // _NBUF


def _sc_gather_all(users, items, neg_flat,
                   U_mlp, U_mf, U_mlp_g, U_mf_g, V_mlp, V_mf):
  mesh = plsc.VectorSubcoreMesh(core_axis_name="c", subcore_axis_name="s")

  out_type = tuple(
      jax.ShapeDtypeStruct((_B, _EMB), jnp.float32) for _ in range(6)
  ) + (
      jax.ShapeDtypeStruct((_NROWS_NEG, _EMB), jnp.float32),
      jax.ShapeDtypeStruct((_NROWS_NEG, _EMB), jnp.float32),
  )

  scratch = (
      [
          pltpu.VMEM((_UB,), jnp.int32),
          pltpu.VMEM((_UB,), jnp.int32),
          pltpu.VMEM((_NNEG * _UB,), jnp.int32),
          pltpu.VMEM((_UB, _EMB), jnp.float32),
      ]
      + [pltpu.VMEM((_UB, _EMB), jnp.float32)] * (2 * _NBUF)
      + [pltpu.SemaphoreType.DMA] * (4 * _NBUF + 1)
  )

  def body(users_h, items_h, neg_h, Um_h, Uf_h, Umg_h, Ufg_h, Vm_h, Vf_h,
           o_um, o_uf, o_umg, o_ufg, o_vm, o_vf, o_nm, o_nf,
           idx_u, idx_i, idx_all, ubuf, *rest):
    bufA = rest[0:_NBUF]
    bufB = rest[_NBUF:2 * _NBUF]
    gA = rest[2 * _NBUF:3 * _NBUF]
    gB = rest[3 * _NBUF:4 * _NBUF]
    sA = rest[4 * _NBUF:5 * _NBUF]
    sB = rest[5 * _NBUF:6 * _NBUF]
    sem_u = rest[6 * _NBUF]

    wid = lax.axis_index("s") * _NC + lax.axis_index("c")
    b0 = wid * _UB

    pltpu.sync_copy(users_h.at[pl.ds(b0, _UB)], idx_u)
    pltpu.sync_copy(items_h.at[pl.ds(b0, _UB)], idx_i)
    pltpu.async_copy(Uf_h.at[idx_u], ubuf, sem_u)
    six = list(zip(
        (Um_h, Uf_h, Umg_h, Ufg_h, Vm_h, Vf_h),
        (o_um, o_uf, o_umg, o_ufg, o_vm, o_vf),
        (idx_u, idx_u, idx_u, idx_u, idx_i, idx_i)))
    bufs4, sems4 = bufA + bufB, gA + gB
    for lo in (0, 4):
      wave = [(t, o, i, bufs4[k], sems4[k])
              for k, (t, o, i) in enumerate(six[lo:lo + 4])]
      for tbl, _, idx, buf, sem in wave:
        pltpu.async_copy(tbl.at[idx], buf, sem)
      for tbl, _, idx, buf, sem in wave:
        pltpu.make_async_copy(tbl.at[idx], buf, sem).wait()
      for _, out, _, buf, sem in wave:
        pltpu.async_copy(buf, out.at[pl.ds(b0, _UB)], sem)
      for _, out, _, buf, sem in wave:
        pltpu.make_async_copy(buf, out.at[pl.ds(b0, _UB)], sem).wait()
    pltpu.make_async_copy(Uf_h.at[idx_u], ubuf, sem_u).wait()

    for n in range(_NNEG):
      pltpu.async_copy(neg_h.at[pl.ds(n * _B + b0, _UB)],
                       idx_all.at[pl.ds(n * _UB, _UB)], sem_u)
    for n in range(_NNEG):
      pltpu.make_async_copy(neg_h.at[pl.ds(n * _B + b0, _UB)],
                            idx_all.at[pl.ds(n * _UB, _UB)], sem_u).wait()

    def g_start(c, j):
      idx = idx_all.at[pl.ds(c * _UB, _UB)]
      pltpu.async_copy(Vm_h.at[idx], bufA[j], gA[j])
      pltpu.async_copy(Vf_h.at[idx], bufB[j], gB[j])

    def g_wait_A(c, j):
      idx = idx_all.at[pl.ds(c * _UB, _UB)]
      pltpu.make_async_copy(Vm_h.at[idx], bufA[j], gA[j]).wait()

    def g_wait_B(c, j):
      idx = idx_all.at[pl.ds(c * _UB, _UB)]
      pltpu.make_async_copy(Vf_h.at[idx], bufB[j], gB[j]).wait()

    def s_start_A(base, j):
      pltpu.async_copy(bufA[j], o_nm.at[pl.ds(base, _UB)], sA[j])

    def s_start_B(base, j):
      pltpu.async_copy(bufB[j], o_nf.at[pl.ds(base, _UB)], sB[j])

    def s_wait(base, j):
      pltpu.make_async_copy(bufA[j], o_nm.at[pl.ds(base, _UB)], sA[j]).wait()
      pltpu.make_async_copy(bufB[j], o_nf.at[pl.ds(base, _UB)], sB[j]).wait()

    def mul_rows(j):
      buf = bufB[j]
      def row(i, carry):
        for k in range(_EMB // 16):
          s = pl.ds(k * 16, 16)
          buf[i, s] = buf[i, s] * ubuf[i, s]
        return carry
      lax.fori_loop(0, _UB, row, 0)

    g_start(0, 0)

    def step(t, carry):
      for j in range(_NBUF):
        c = t * _NBUF + j
        base = c * _B + b0
        jn = (j - 1) % _NBUF
        g_wait_A(c, j)
        s_start_A(base, j)
        g_wait_B(c, j)
        if j == 0:
          @pl.when(t > 0)
          def _():
            s_wait(base - _B, jn)
          g_start(c + 1, jn)
        else:
          s_wait(base - _B, jn)
          @pl.when(t < _T - 1)
          def _():
            g_start(c + 1, jn)
        mul_rows(j)
        s_start_B(base, j)
      return carry

    lax.fori_loop(0, _T, step, 0)
    s_wait((_NNEG - 1) * _B + b0, _NBUF - 1)

  return functools.partial(
      pl.kernel, mesh=mesh, out_type=out_type,
      compiler_params=pltpu.CompilerParams(use_tc_tiling_on_sc=True),
      scratch_types=scratch,
  )(body)(users, items, neg_flat, U_mlp, U_mf, U_mlp_g, U_mf_g, V_mlp, V_mf)


def _tc_mlp_kernel(u_mlp_r, v_mlp_r, u_mf_r, v_mf_r, nvm_r,
                   W1u_r, W1v_r, b1_r, W2_r, b2_r,
                   o_mlp, o_mf, o_nmlp, preu_s):
  f32 = jnp.float32
  bf16 = jnp.bfloat16
  dot = lambda a, b: jax.lax.dot_general(
      a.astype(bf16), b, (((1,), (0,)), ((), ())), preferred_element_type=f32)
  i = pl.program_id(0)
  W1v = W1v_r[...].astype(bf16)
  W2 = W2_r[...].astype(bf16)

  @pl.when(i == 0)
  def _():
    pre_u = dot(u_mlp_r[...], W1u_r[...].astype(bf16)) + b1_r[...]
    preu_s[...] = pre_u
    pre_v = dot(v_mlp_r[...], W1v)
    h = jax.nn.relu(pre_u + pre_v)
    o_mlp[...] = dot(h, W2) + b2_r[...]
    o_mf[...] = u_mf_r[...] * v_mf_r[...]

  @pl.when(i > 0)
  def _():
    pre_nv = dot(nvm_r[...], W1v)
    h = jax.nn.relu(preu_s[...] + pre_nv)
    o_nmlp[...] = dot(h, W2) + b2_r[...]


def _tc_mlp(u_mlp, v_mlp, u_mf, v_mf, nm, W1u, W1v, b1, W2, b2):
  blk0 = pl.BlockSpec((_B, _EMB), lambda i: (0, 0))
  nblk = pl.BlockSpec((_B, _EMB), lambda i: (jnp.maximum(i - 1, 0), 0))
  rep = lambda r: pl.BlockSpec((r, _EMB), lambda i: (0, 0))
  return pl.pallas_call(
      _tc_mlp_kernel,
      grid=(_NNEG + 1,),
      in_specs=[blk0, blk0, blk0, blk0, nblk,
                rep(_EMB), rep(_EMB), rep(1), rep(_EMB), rep(1)],
      out_specs=[blk0, blk0, nblk],
      out_shape=[
          jax.ShapeDtypeStruct((_B, _EMB), jnp.float32),
          jax.ShapeDtypeStruct((_B, _EMB), jnp.float32),
          jax.ShapeDtypeStruct((_NROWS_NEG, _EMB), jnp.float32),
      ],
      scratch_shapes=[pltpu.VMEM((_B, _EMB), jnp.float32)],
  )(u_mlp, v_mlp, u_mf, v_mf, nm, W1u, W1v, b1, W2, b2)


def kernel(users, items, neg_items, U_mlp, U_mf, V_mlp, V_mf,
           U_mlp_g, U_mf_g, W1, b1, W2, b2):
  users = users.astype(jnp.int32)
  items = items.astype(jnp.int32)
  neg_flat = neg_items.astype(jnp.int32).T.reshape(-1)

  (u_mlp, u_mf, u_mlp_g, u_mf_g, v_mlp, v_mf, nm, neg_mf_flat) = \
      _sc_gather_all(users, items, neg_flat, U_mlp, U_mf, U_mlp_g, U_mf_g,
                     V_mlp, V_mf)

  W1u = W1[:_EMB]
  W1v = W1[_EMB:]
  b1r = b1.reshape(1, _EMB)
  b2r = b2.reshape(1, _EMB)

  mlp_vec, mf_vec, neg_mlp = _tc_mlp(
      u_mlp, v_mlp, u_mf, v_mf, nm, W1u, W1v, b1r, W2, b2r)

  neg_mlp = jnp.transpose(neg_mlp.reshape(_NNEG, _B, _EMB), (1, 0, 2))
  neg_mf = jnp.transpose(neg_mf_flat.reshape(_NNEG, _B, _EMB), (1, 0, 2))

  return (mlp_vec, mf_vec, u_mlp, u_mf, u_mlp_g, u_mf_g, neg_mlp, neg_mf)
